# baseline (speedup 1.0000x reference)
import functools

import jax
import jax.numpy as jnp
from jax import lax
from jax.experimental import pallas as pl
from jax.experimental.pallas import tpu as pltpu
from jax.experimental.pallas import tpu_sc as plsc

_H = 512
_W = 512
_N = _H * _W
_KLO = 26214
_KMAX = 26214
_KHI = _N - _KMAX + 1
_IGNORE = 255

_NW = 32
_CH = 16384
_NCH = _N // _CH
_NB1 = 2048
_NB3 = 1024
_INT_MIN = -2147483648


def _scan_hist(ref, base, nrows, k):
    ngroups = nrows // 16

    def gbody(g, carry):
        cum, bsel, cbel, found = carry

        def rbody(j, gs):
            return gs + ref[base + g * 16 + j]

        gsum = lax.fori_loop(0, 16, rbody, jnp.zeros((16,), jnp.int32))
        gt = jnp.sum(gsum)
        crossed = (1 - found) * jnp.where(cum + gt >= k, 1, 0)
        bsel = jnp.where(crossed == 1, g, bsel)
        cbel = jnp.where(crossed == 1, cum, cbel)
        return cum + gt, bsel, cbel, found | crossed

    zero = jnp.int32(0)
    _, gstar, cbase, _ = lax.fori_loop(0, ngroups, gbody,
                                       (zero, zero, zero, zero))

    def jbody(j, carry):
        cum, bsel, cbel, found = carry
        t = jnp.sum(ref[base + gstar * 16 + j])
        crossed = (1 - found) * jnp.where(cum + t >= k, 1, 0)
        bsel = jnp.where(crossed == 1, j, bsel)
        cbel = jnp.where(crossed == 1, cum, cbel)
        return cum + t, bsel, cbel, found | crossed

    _, jstar, cbel, _ = lax.fori_loop(0, 16, jbody, (cbase, zero, cbase, zero))
    return gstar * 16 + jstar, cbel


def _sc_select_body(x_hbm, out_hbm, chunk, hist_a, hist_b, hist_c, outv):
    w = lax.axis_index("s") * 2 + lax.axis_index("c")
    lanes = lax.iota(jnp.int32, 16)
    ones = jnp.full((16,), 1, jnp.int32)

    def zero_rows(ref, nrows):
        def zb(b, c):
            ref[b] = jnp.zeros((16,), jnp.int32)
            return c
        lax.fori_loop(0, nrows, zb, 0)

    def data_pass(process_vreg):
        def chunk_body(g, carry):
            pltpu.sync_copy(x_hbm.at[w, pl.ds(g * _CH, _CH)], chunk)

            def vbody(i, c2):
                v = chunk[pl.ds(i * 16, 16)]
                process_vreg(v)
                return c2

            lax.fori_loop(0, _CH // 16, vbody, 0)
            return carry

        lax.fori_loop(0, _NCH, chunk_body, 0)

    def ukey_of(s):
        key = jnp.where(s >= 0, s, s ^ jnp.int32(0x7FFFFFFF))
        return key ^ jnp.int32(_INT_MIN)

    zero_rows(hist_a, _NB1)
    zero_rows(hist_b, _NB1)
    zero_rows(hist_c, _NB1)

    def p1(v):
        b1 = lax.shift_right_logical(ukey_of(v), 21)
        plsc.addupdate_scatter(hist_a, [b1, lanes], ones)

    data_pass(p1)

    b1_lo, c1_lo = _scan_hist(hist_a, 0, _NB1, jnp.int32(_KLO))
    b1_hi, c1_hi = _scan_hist(hist_a, 0, _NB1, jnp.int32(_KHI))
    r_lo = jnp.int32(_KLO) - c1_lo
    r_hi = jnp.int32(_KHI) - c1_hi

    zero_rows(hist_a, _NB1)

    def p2(v):
        u = ukey_of(v)
        b1 = lax.shift_right_logical(u, 21)
        b2 = lax.shift_right_logical(u, 10) & jnp.int32(0x7FF)
        plsc.addupdate_scatter(hist_b, [b2, lanes], ones, mask=(b1 == b1_lo))
        plsc.addupdate_scatter(hist_c, [b2, lanes], ones, mask=(b1 == b1_hi))

    data_pass(p2)

    b2_lo, c2_lo = _scan_hist(hist_b, 0, _NB1, r_lo)
    b2_hi, c2_hi = _scan_hist(hist_c, 0, _NB1, r_hi)
    r2_lo = r_lo - c2_lo
    r2_hi = r_hi - c2_hi

    p22_lo = b1_lo * 2048 + b2_lo
    p22_hi = b1_hi * 2048 + b2_hi

    def p3(v):
        u = ukey_of(v)
        p22 = lax.shift_right_logical(u, 10)
        b3 = u & jnp.int32(0x3FF)
        plsc.addupdate_scatter(hist_a, [b3, lanes], ones, mask=(p22 == p22_lo))
        plsc.addupdate_scatter(hist_a, [b3 + jnp.int32(_NB3), lanes], ones,
                               mask=(p22 == p22_hi))

    data_pass(p3)

    b3_lo, _ = _scan_hist(hist_a, 0, _NB3, r2_lo)
    b3_hi, _ = _scan_hist(hist_a, _NB3, _NB3, r2_hi)

    ukey_lo = (p22_lo << 10) | b3_lo
    ukey_hi = (p22_hi << 10) | b3_hi
    key_lo = ukey_lo ^ jnp.int32(_INT_MIN)
    key_hi = ukey_hi ^ jnp.int32(_INT_MIN)

    outv[...] = jnp.where(lanes == 0, key_lo,
                          jnp.where(lanes == 1, key_hi, jnp.int32(0)))
    pltpu.sync_copy(outv, out_hbm.at[w])


def _label_body(thr_ref, x_ref, out_ref):
    i = pl.program_id(0)
    qlo = thr_ref[i, 0]
    qhi = thr_ref[i, 1]
    qlo = jnp.where(qlo == jnp.int32(-1), jnp.int32(0), qlo)
    qhi = jnp.where(qhi == jnp.int32(-1), jnp.int32(0), qhi)

    x = x_ref[0, 0]
    s = lax.bitcast_convert_type(x, jnp.int32)
    key = jnp.where(s >= 0, s, s ^ jnp.int32(0x7FFFFFFF))
    key = jnp.where(s == jnp.int32(_INT_MIN), jnp.int32(0), key)

    m_lo = jnp.sum((key < qlo).astype(jnp.int32))
    m_hi = jnp.sum((key < qhi).astype(jnp.int32))

    eq_lo = (key == qlo)
    eq_hi = (key == qhi)

    r_iota = lax.broadcasted_iota(jnp.int32, (_H, _W), 0)
    c_iota = lax.broadcasted_iota(jnp.int32, (_H, _W), 1)
    upper = (r_iota < c_iota).astype(jnp.bfloat16)
    lower = (c_iota < r_iota).astype(jnp.float32)

    dn = (((1,), (0,)), ((), ()))

    def prefix_of(eq):
        eq_b = eq.astype(jnp.bfloat16)
        within = lax.dot_general(eq_b, upper, dn,
                                 preferred_element_type=jnp.float32)
        rowsum = jnp.sum(eq.astype(jnp.float32), axis=1, keepdims=True)
        row_prefix = lax.dot_general(lower, rowsum, dn,
                                     preferred_element_type=jnp.float32)
        return within + row_prefix

    pref_lo = prefix_of(eq_lo)
    pref_hi = prefix_of(eq_hi)

    t_lo = (_KLO - m_lo).astype(jnp.float32)
    t_hi = (jnp.int32(_N) - m_hi - jnp.int32(_KMAX)).astype(jnp.float32)
    bg = (key < qlo) | (eq_lo & (pref_lo < t_lo))
    fg = (key > qhi) | (eq_hi & (pref_hi >= t_hi))

    out = jnp.where(fg, jnp.int32(1),
                    jnp.where(bg, jnp.int32(0), jnp.int32(_IGNORE)))
    out_ref[0] = out


def _sc_select(xf):
    mesh = plsc.VectorSubcoreMesh(core_axis_name="c", subcore_axis_name="s")
    return pl.kernel(
        _sc_select_body,
        out_type=jax.ShapeDtypeStruct((_NW, 16), jnp.int32),
        mesh=mesh,
        compiler_params=pltpu.CompilerParams(needs_layout_passes=False,
                                             use_tc_tiling_on_sc=False),
        scratch_types=[
            pltpu.VMEM((_CH,), jnp.int32),
            pltpu.VMEM((_NB1, 16), jnp.int32),
            pltpu.VMEM((_NB1, 16), jnp.int32),
            pltpu.VMEM((_NB1, 16), jnp.int32),
            pltpu.VMEM((16,), jnp.int32),
        ],
    )(xf)


def _tc_label(thr, x):
    b = x.shape[0]
    grid_spec = pl.GridSpec(
        grid=(b,),
        in_specs=[
            pl.BlockSpec(memory_space=pltpu.SMEM),
            pl.BlockSpec((1, 1, _H, _W), lambda i: (i, 0, 0, 0)),
        ],
        out_specs=pl.BlockSpec((1, _H, _W), lambda i: (i, 0, 0)),
    )
    return pl.pallas_call(
        _label_body,
        grid_spec=grid_spec,
        out_shape=jax.ShapeDtypeStruct((b, _H, _W), jnp.int32),
    )(thr, x)


@jax.jit
def kernel(x):
    b = x.shape[0]
    xf = lax.bitcast_convert_type(x.reshape(b, _N), jnp.int32)
    thr = _sc_select(xf)
    return _tc_label(thr, x)


# speedup vs baseline: 1.3241x; 1.3241x over previous
import functools

import jax
import jax.numpy as jnp
from jax import lax
from jax.experimental import pallas as pl
from jax.experimental.pallas import tpu as pltpu
from jax.experimental.pallas import tpu_sc as plsc

_H = 512
_W = 512
_N = _H * _W
_KLO = 26214
_KMAX = 26214
_KHI = _N - _KMAX + 1
_IGNORE = 255

_NW = 32
_CH = 8192
_NCH = _N // _CH
_NSUP = _NCH // 2
_NB1 = 2048
_NB3 = 1024
_INT_MIN = -2147483648
_UNROLL = 8


def _scan_hist2(ref, base, nrows, k_a, k_b, row_of):
    ngroups = nrows // 16
    zero = jnp.int32(0)

    def gbody(g, carry):
        cum, ga, ca, fa, gb, cb, fb = carry

        def rbody(j, gs):
            return gs + ref[base + row_of(g * 16 + j)]

        gsum = lax.fori_loop(0, 16, rbody, jnp.zeros((16,), jnp.int32))
        gt = jnp.sum(gsum)
        nxt = cum + gt
        cross_a = (1 - fa) * jnp.where(nxt >= k_a, 1, 0)
        ga = jnp.where(cross_a == 1, g, ga)
        ca = jnp.where(cross_a == 1, cum, ca)
        cross_b = (1 - fb) * jnp.where(nxt >= k_b, 1, 0)
        gb = jnp.where(cross_b == 1, g, gb)
        cb = jnp.where(cross_b == 1, cum, cb)
        return nxt, ga, ca, fa | cross_a, gb, cb, fb | cross_b

    _, ga, ca, _, gb, cb, _ = lax.fori_loop(
        0, ngroups, gbody, (zero, zero, zero, zero, zero, zero, zero))

    def drill(gstar, cbase, k):
        def jbody(j, carry):
            cum, bsel, cbel, found = carry
            t = jnp.sum(ref[base + row_of(gstar * 16 + j)])
            crossed = (1 - found) * jnp.where(cum + t >= k, 1, 0)
            bsel = jnp.where(crossed == 1, j, bsel)
            cbel = jnp.where(crossed == 1, cum, cbel)
            return cum + t, bsel, cbel, found | crossed

        _, jstar, cbel, _ = lax.fori_loop(0, 16, jbody,
                                          (cbase, zero, cbase, zero))
        return row_of(gstar * 16 + jstar), cbel

    ba, ca2 = drill(ga, ca, k_a)
    bb, cb2 = drill(gb, cb, k_b)
    return ba, ca2, bb, cb2


def _sc_select_body(x_hbm, out_hbm, buf0, buf1, hist_a, hist_b, hist_c, outv,
                    sem0, sem1):
    w = lax.axis_index("s") * 2 + lax.axis_index("c")
    lanes = lax.iota(jnp.int32, 16)
    ones = jnp.full((16,), 1, jnp.int32)

    def zero_rows(ref, nrows):
        def zb(b, c):
            for u in range(_UNROLL):
                ref[b * _UNROLL + u] = jnp.zeros((16,), jnp.int32)
            return c
        lax.fori_loop(0, nrows // _UNROLL, zb, 0)

    def data_pass(process_vreg):
        def inner(buf):
            def vbody(i, c2):
                for u in range(_UNROLL):
                    process_vreg(buf[pl.ds(i * 16 * _UNROLL + u * 16, 16)])
                return c2
            lax.fori_loop(0, _CH // (16 * _UNROLL), vbody, 0)

        def wait(buf, sem):
            pltpu.make_async_copy(x_hbm.at[w, pl.ds(0, _CH)], buf, sem).wait()

        pltpu.async_copy(x_hbm.at[w, pl.ds(0, _CH)], buf0, sem0)

        def super_body(sg, carry):
            g0 = sg * 2
            pltpu.async_copy(x_hbm.at[w, pl.ds((g0 + 1) * _CH, _CH)],
                             buf1, sem1)
            wait(buf0, sem0)
            inner(buf0)

            @pl.when(sg < _NSUP - 1)
            def _():
                pltpu.async_copy(x_hbm.at[w, pl.ds((g0 + 2) * _CH, _CH)],
                                 buf0, sem0)

            wait(buf1, sem1)
            inner(buf1)
            return carry

        lax.fori_loop(0, _NSUP, super_body, 0)

    zero_rows(hist_a, _NB1)
    zero_rows(hist_b, _NB1)
    zero_rows(hist_c, _NB1)

    def p1(v):
        b1 = lax.shift_right_logical(v, 21)
        plsc.addupdate_scatter(hist_a, [b1, lanes], ones)

    data_pass(p1)

    def l1_row(l):
        return jnp.where(l < 1024, 2047 - l, l - 1024)

    b1_lo, c1_lo, b1_hi, c1_hi = _scan_hist2(
        hist_a, 0, _NB1, jnp.int32(_KLO), jnp.int32(_KHI), l1_row)
    r_lo = jnp.int32(_KLO) - c1_lo
    r_hi = jnp.int32(_KHI) - c1_hi
    neg_lo = jnp.where(b1_lo >= 1024, 1, 0)
    neg_hi = jnp.where(b1_hi >= 1024, 1, 0)

    zero_rows(hist_a, _NB1)

    def p2(v):
        b1 = lax.shift_right_logical(v, 21)
        b2 = lax.shift_right_logical(v, 10) & jnp.int32(0x7FF)
        plsc.addupdate_scatter(hist_b, [b2, lanes], ones, mask=(b1 == b1_lo))
        plsc.addupdate_scatter(hist_c, [b2, lanes], ones, mask=(b1 == b1_hi))

    data_pass(p2)

    def flip_row(neg, nrows):
        def row_of(l):
            return jnp.where(neg == 1, nrows - 1 - l, l)
        return row_of

    b2_lo, c2_lo, _, _ = _scan_hist2(hist_b, 0, _NB1, r_lo, r_lo,
                                     flip_row(neg_lo, _NB1))
    b2_hi, c2_hi, _, _ = _scan_hist2(hist_c, 0, _NB1, r_hi, r_hi,
                                     flip_row(neg_hi, _NB1))
    r2_lo = r_lo - c2_lo
    r2_hi = r_hi - c2_hi

    p22_lo = b1_lo * 2048 + b2_lo
    p22_hi = b1_hi * 2048 + b2_hi

    def p3(v):
        p22 = lax.shift_right_logical(v, 10)
        b3 = v & jnp.int32(0x3FF)
        plsc.addupdate_scatter(hist_a, [b3, lanes], ones, mask=(p22 == p22_lo))
        plsc.addupdate_scatter(hist_a, [b3 + jnp.int32(_NB3), lanes], ones,
                               mask=(p22 == p22_hi))

    data_pass(p3)

    b3_lo, _, _, _ = _scan_hist2(hist_a, 0, _NB3, r2_lo, r2_lo,
                                 flip_row(neg_lo, _NB3))
    b3_hi, _, _, _ = _scan_hist2(hist_a, _NB3, _NB3, r2_hi, r2_hi,
                                 flip_row(neg_hi, _NB3))

    raw_lo = (p22_lo << 10) | b3_lo
    raw_hi = (p22_hi << 10) | b3_hi
    key_lo = jnp.where(raw_lo >= 0, raw_lo, raw_lo ^ jnp.int32(0x7FFFFFFF))
    key_hi = jnp.where(raw_hi >= 0, raw_hi, raw_hi ^ jnp.int32(0x7FFFFFFF))

    outv[...] = jnp.where(lanes == 0, key_lo,
                          jnp.where(lanes == 1, key_hi, jnp.int32(0)))
    pltpu.sync_copy(outv, out_hbm.at[w])


def _label_body(thr_ref, x_ref, out_ref):
    i = pl.program_id(0)
    qlo = thr_ref[i, 0]
    qhi = thr_ref[i, 1]
    qlo = jnp.where(qlo == jnp.int32(-1), jnp.int32(0), qlo)
    qhi = jnp.where(qhi == jnp.int32(-1), jnp.int32(0), qhi)

    x = x_ref[0, 0]
    s = lax.bitcast_convert_type(x, jnp.int32)
    key = jnp.where(s >= 0, s, s ^ jnp.int32(0x7FFFFFFF))
    key = jnp.where(s == jnp.int32(_INT_MIN), jnp.int32(0), key)

    m_lo = jnp.sum((key < qlo).astype(jnp.int32))
    m_hi = jnp.sum((key < qhi).astype(jnp.int32))

    eq_lo = (key == qlo)
    eq_hi = (key == qhi)

    r_iota = lax.broadcasted_iota(jnp.int32, (_H, _W), 0)
    c_iota = lax.broadcasted_iota(jnp.int32, (_H, _W), 1)
    upper = (r_iota < c_iota).astype(jnp.bfloat16)
    lower = (c_iota < r_iota).astype(jnp.float32)

    dn = (((1,), (0,)), ((), ()))

    def prefix_of(eq):
        eq_b = eq.astype(jnp.bfloat16)
        within = lax.dot_general(eq_b, upper, dn,
                                 preferred_element_type=jnp.float32)
        rowsum = jnp.sum(eq.astype(jnp.float32), axis=1, keepdims=True)
        row_prefix = lax.dot_general(lower, rowsum, dn,
                                     preferred_element_type=jnp.float32)
        return within + row_prefix

    pref_lo = prefix_of(eq_lo)
    pref_hi = prefix_of(eq_hi)

    t_lo = (_KLO - m_lo).astype(jnp.float32)
    t_hi = (jnp.int32(_N) - m_hi - jnp.int32(_KMAX)).astype(jnp.float32)
    bg = (key < qlo) | (eq_lo & (pref_lo < t_lo))
    fg = (key > qhi) | (eq_hi & (pref_hi >= t_hi))

    out = jnp.where(fg, jnp.int32(1),
                    jnp.where(bg, jnp.int32(0), jnp.int32(_IGNORE)))
    out_ref[0] = out


def _sc_select(xf):
    mesh = plsc.VectorSubcoreMesh(core_axis_name="c", subcore_axis_name="s")
    return pl.kernel(
        _sc_select_body,
        out_type=jax.ShapeDtypeStruct((_NW, 16), jnp.int32),
        mesh=mesh,
        compiler_params=pltpu.CompilerParams(needs_layout_passes=False,
                                             use_tc_tiling_on_sc=False),
        scratch_types=[
            pltpu.VMEM((_CH,), jnp.int32),
            pltpu.VMEM((_CH,), jnp.int32),
            pltpu.VMEM((_NB1, 16), jnp.int32),
            pltpu.VMEM((_NB1, 16), jnp.int32),
            pltpu.VMEM((_NB1, 16), jnp.int32),
            pltpu.VMEM((16,), jnp.int32),
            pltpu.SemaphoreType.DMA,
            pltpu.SemaphoreType.DMA,
        ],
    )(xf)


def _tc_label(thr, x):
    b = x.shape[0]
    grid_spec = pl.GridSpec(
        grid=(b,),
        in_specs=[
            pl.BlockSpec(memory_space=pltpu.SMEM),
            pl.BlockSpec((1, 1, _H, _W), lambda i: (i, 0, 0, 0)),
        ],
        out_specs=pl.BlockSpec((1, _H, _W), lambda i: (i, 0, 0)),
    )
    return pl.pallas_call(
        _label_body,
        grid_spec=grid_spec,
        out_shape=jax.ShapeDtypeStruct((b, _H, _W), jnp.int32),
    )(thr, x)


@jax.jit
def kernel(x):
    b = x.shape[0]
    xf = lax.bitcast_convert_type(x.reshape(b, _N), jnp.int32)
    thr = _sc_select(xf)
    return _tc_label(thr, x)


# speedup vs baseline: 2.7856x; 2.1038x over previous
import functools

import jax
import jax.numpy as jnp
from jax import lax
from jax.experimental import pallas as pl
from jax.experimental.pallas import tpu as pltpu
from jax.experimental.pallas import tpu_sc as plsc

_H = 512
_W = 512
_N = _H * _W
_KLO = 26214
_KMAX = 26214
_KHI = _N - _KMAX + 1
_IGNORE = 255

_NW = 32
_CH = 8192
_NCH = _N // _CH
_NSUP = _NCH // 2
_NB1 = 2048
_NB3 = 1024
_INT_MIN = -2147483648
_UNROLL = 8


def _scan_hist2(ref, base, nrows, k_a, k_b, row_of):
    ngroups = nrows // 16
    zero = jnp.int32(0)

    def gbody(g, carry):
        cum, ga, ca, fa, gb, cb, fb = carry

        def rbody(j, gs):
            return gs + ref[base + row_of(g * 16 + j)]

        gsum = lax.fori_loop(0, 16, rbody, jnp.zeros((16,), jnp.int32))
        gt = jnp.sum(gsum)
        nxt = cum + gt
        cross_a = (1 - fa) * jnp.where(nxt >= k_a, 1, 0)
        ga = jnp.where(cross_a == 1, g, ga)
        ca = jnp.where(cross_a == 1, cum, ca)
        cross_b = (1 - fb) * jnp.where(nxt >= k_b, 1, 0)
        gb = jnp.where(cross_b == 1, g, gb)
        cb = jnp.where(cross_b == 1, cum, cb)
        return nxt, ga, ca, fa | cross_a, gb, cb, fb | cross_b

    _, ga, ca, _, gb, cb, _ = lax.fori_loop(
        0, ngroups, gbody, (zero, zero, zero, zero, zero, zero, zero))

    def drill(gstar, cbase, k):
        def jbody(j, carry):
            cum, bsel, cbel, found = carry
            t = jnp.sum(ref[base + row_of(gstar * 16 + j)])
            crossed = (1 - found) * jnp.where(cum + t >= k, 1, 0)
            bsel = jnp.where(crossed == 1, j, bsel)
            cbel = jnp.where(crossed == 1, cum, cbel)
            return cum + t, bsel, cbel, found | crossed

        _, jstar, cbel, _ = lax.fori_loop(0, 16, jbody,
                                          (cbase, zero, cbase, zero))
        return row_of(gstar * 16 + jstar), cbel

    ba, ca2 = drill(ga, ca, k_a)
    bb, cb2 = drill(gb, cb, k_b)
    return ba, ca2, bb, cb2


def _sc_select_body(x_hbm, out_hbm, buf0, buf1, hist_a, hist_b, hist_c, outv,
                    sem0, sem1):
    w = lax.axis_index("s") * 2 + lax.axis_index("c")
    lanes = lax.iota(jnp.int32, 16)
    ones = jnp.full((16,), 1, jnp.int32)

    def zero_rows(ref, nrows):
        def zb(b, c):
            for u in range(_UNROLL):
                ref[b * _UNROLL + u] = jnp.zeros((16,), jnp.int32)
            return c
        lax.fori_loop(0, nrows // _UNROLL, zb, 0)

    def data_pass(process_vregs):
        def inner(buf):
            def vbody(i, c2):
                base = i * 16 * _UNROLL
                vs = [buf[pl.ds(base + u * 16, 16)] for u in range(_UNROLL)]
                process_vregs(vs)
                return c2
            lax.fori_loop(0, _CH // (16 * _UNROLL), vbody, 0)

        def wait(buf, sem):
            pltpu.make_async_copy(x_hbm.at[w, pl.ds(0, _CH)], buf, sem).wait()

        pltpu.async_copy(x_hbm.at[w, pl.ds(0, _CH)], buf0, sem0)

        def super_body(sg, carry):
            g0 = sg * 2
            pltpu.async_copy(x_hbm.at[w, pl.ds((g0 + 1) * _CH, _CH)],
                             buf1, sem1)
            wait(buf0, sem0)
            inner(buf0)

            @pl.when(sg < _NSUP - 1)
            def _():
                pltpu.async_copy(x_hbm.at[w, pl.ds((g0 + 2) * _CH, _CH)],
                                 buf0, sem0)

            wait(buf1, sem1)
            inner(buf1)
            return carry

        lax.fori_loop(0, _NSUP, super_body, 0)

    zero_rows(hist_a, _NB1)
    zero_rows(hist_b, _NB1)
    zero_rows(hist_c, _NB1)

    def p1(vs):
        b1s = [lax.shift_right_logical(v, 21) for v in vs]
        for b1 in b1s:
            plsc.addupdate_scatter(hist_a, [b1, lanes], ones)

    data_pass(p1)

    def l1_row(l):
        return jnp.where(l < 1024, 2047 - l, l - 1024)

    b1_lo, c1_lo, b1_hi, c1_hi = _scan_hist2(
        hist_a, 0, _NB1, jnp.int32(_KLO), jnp.int32(_KHI), l1_row)
    r_lo = jnp.int32(_KLO) - c1_lo
    r_hi = jnp.int32(_KHI) - c1_hi
    neg_lo = jnp.where(b1_lo >= 1024, 1, 0)
    neg_hi = jnp.where(b1_hi >= 1024, 1, 0)

    zero_rows(hist_a, _NB1)

    def p2(vs):
        b1s = [lax.shift_right_logical(v, 21) for v in vs]
        b2s = [lax.shift_right_logical(v, 10) & jnp.int32(0x7FF) for v in vs]
        m_los = [b1 == b1_lo for b1 in b1s]
        m_his = [b1 == b1_hi for b1 in b1s]
        for b2, m in zip(b2s, m_los):
            plsc.addupdate_scatter(hist_b, [b2, lanes], ones, mask=m)
        for b2, m in zip(b2s, m_his):
            plsc.addupdate_scatter(hist_c, [b2, lanes], ones, mask=m)

    data_pass(p2)

    def flip_row(neg, nrows):
        def row_of(l):
            return jnp.where(neg == 1, nrows - 1 - l, l)
        return row_of

    b2_lo, c2_lo, _, _ = _scan_hist2(hist_b, 0, _NB1, r_lo, r_lo,
                                     flip_row(neg_lo, _NB1))
    b2_hi, c2_hi, _, _ = _scan_hist2(hist_c, 0, _NB1, r_hi, r_hi,
                                     flip_row(neg_hi, _NB1))
    r2_lo = r_lo - c2_lo
    r2_hi = r_hi - c2_hi

    p22_lo = b1_lo * 2048 + b2_lo
    p22_hi = b1_hi * 2048 + b2_hi

    def p3(vs):
        p22s = [lax.shift_right_logical(v, 10) for v in vs]
        b3s = [v & jnp.int32(0x3FF) for v in vs]
        m_los = [p22 == p22_lo for p22 in p22s]
        m_his = [p22 == p22_hi for p22 in p22s]
        for b3, m in zip(b3s, m_los):
            plsc.addupdate_scatter(hist_a, [b3, lanes], ones, mask=m)
        for b3, m in zip(b3s, m_his):
            plsc.addupdate_scatter(hist_a, [b3 + jnp.int32(_NB3), lanes],
                                   ones, mask=m)

    data_pass(p3)

    b3_lo, _, _, _ = _scan_hist2(hist_a, 0, _NB3, r2_lo, r2_lo,
                                 flip_row(neg_lo, _NB3))
    b3_hi, _, _, _ = _scan_hist2(hist_a, _NB3, _NB3, r2_hi, r2_hi,
                                 flip_row(neg_hi, _NB3))

    raw_lo = (p22_lo << 10) | b3_lo
    raw_hi = (p22_hi << 10) | b3_hi
    key_lo = jnp.where(raw_lo >= 0, raw_lo, raw_lo ^ jnp.int32(0x7FFFFFFF))
    key_hi = jnp.where(raw_hi >= 0, raw_hi, raw_hi ^ jnp.int32(0x7FFFFFFF))

    outv[...] = jnp.where(lanes == 0, key_lo,
                          jnp.where(lanes == 1, key_hi, jnp.int32(0)))
    pltpu.sync_copy(outv, out_hbm.at[w])


def _label_body(thr_ref, x_ref, out_ref):
    i = pl.program_id(0)
    qlo = thr_ref[i, 0]
    qhi = thr_ref[i, 1]
    qlo = jnp.where(qlo == jnp.int32(-1), jnp.int32(0), qlo)
    qhi = jnp.where(qhi == jnp.int32(-1), jnp.int32(0), qhi)

    x = x_ref[0, 0]
    s = lax.bitcast_convert_type(x, jnp.int32)
    key = jnp.where(s >= 0, s, s ^ jnp.int32(0x7FFFFFFF))
    key = jnp.where(s == jnp.int32(_INT_MIN), jnp.int32(0), key)

    m_lo = jnp.sum((key < qlo).astype(jnp.int32))
    m_hi = jnp.sum((key < qhi).astype(jnp.int32))

    eq_lo = (key == qlo)
    eq_hi = (key == qhi)

    r_iota = lax.broadcasted_iota(jnp.int32, (_H, _W), 0)
    c_iota = lax.broadcasted_iota(jnp.int32, (_H, _W), 1)
    upper = (r_iota < c_iota).astype(jnp.bfloat16)
    lower = (c_iota < r_iota).astype(jnp.float32)

    dn = (((1,), (0,)), ((), ()))

    def prefix_of(eq):
        eq_b = eq.astype(jnp.bfloat16)
        within = lax.dot_general(eq_b, upper, dn,
                                 preferred_element_type=jnp.float32)
        rowsum = jnp.sum(eq.astype(jnp.float32), axis=1, keepdims=True)
        row_prefix = lax.dot_general(lower, rowsum, dn,
                                     preferred_element_type=jnp.float32)
        return within + row_prefix

    pref_lo = prefix_of(eq_lo)
    pref_hi = prefix_of(eq_hi)

    t_lo = (_KLO - m_lo).astype(jnp.float32)
    t_hi = (jnp.int32(_N) - m_hi - jnp.int32(_KMAX)).astype(jnp.float32)
    bg = (key < qlo) | (eq_lo & (pref_lo < t_lo))
    fg = (key > qhi) | (eq_hi & (pref_hi >= t_hi))

    out = jnp.where(fg, jnp.int32(1),
                    jnp.where(bg, jnp.int32(0), jnp.int32(_IGNORE)))
    out_ref[0] = out


def _sc_select(xf):
    mesh = plsc.VectorSubcoreMesh(core_axis_name="c", subcore_axis_name="s")
    return pl.kernel(
        _sc_select_body,
        out_type=jax.ShapeDtypeStruct((_NW, 16), jnp.int32),
        mesh=mesh,
        compiler_params=pltpu.CompilerParams(needs_layout_passes=False,
                                             use_tc_tiling_on_sc=False),
        scratch_types=[
            pltpu.VMEM((_CH,), jnp.int32),
            pltpu.VMEM((_CH,), jnp.int32),
            pltpu.VMEM((_NB1, 16), jnp.int32),
            pltpu.VMEM((_NB1, 16), jnp.int32),
            pltpu.VMEM((_NB1, 16), jnp.int32),
            pltpu.VMEM((16,), jnp.int32),
            pltpu.SemaphoreType.DMA,
            pltpu.SemaphoreType.DMA,
        ],
    )(xf)


def _tc_label(thr, x):
    b = x.shape[0]
    grid_spec = pl.GridSpec(
        grid=(b,),
        in_specs=[
            pl.BlockSpec(memory_space=pltpu.SMEM),
            pl.BlockSpec((1, 1, _H, _W), lambda i: (i, 0, 0, 0)),
        ],
        out_specs=pl.BlockSpec((1, _H, _W), lambda i: (i, 0, 0)),
    )
    return pl.pallas_call(
        _label_body,
        grid_spec=grid_spec,
        out_shape=jax.ShapeDtypeStruct((b, _H, _W), jnp.int32),
    )(thr, x)


@jax.jit
def kernel(x):
    b = x.shape[0]
    xf = lax.bitcast_convert_type(x.reshape(b, _N), jnp.int32)
    thr = _sc_select(xf)
    return _tc_label(thr, x)


# speedup vs baseline: 2.8136x; 1.0100x over previous
import functools

import jax
import jax.numpy as jnp
from jax import lax
from jax.experimental import pallas as pl
from jax.experimental.pallas import tpu as pltpu
from jax.experimental.pallas import tpu_sc as plsc

_H = 512
_W = 512
_N = _H * _W
_KLO = 26214
_KMAX = 26214
_KHI = _N - _KMAX + 1
_IGNORE = 255

_NW = 32
_CH = 8192
_NCH = _N // _CH
_NSUP = _NCH // 2
_NB1 = 2048
_NB3 = 1024
_INT_MIN = -2147483648
_UNROLL = 8


def _scan_hist2(ref, base, nrows, k_a, k_b, row_of):
    ngroups = nrows // 16
    zero = jnp.int32(0)

    def gbody(g, carry):
        cum, ga, ca, fa, gb, cb, fb = carry

        def rbody(j, gs):
            return gs + ref[base + row_of(g * 16 + j)]

        gsum = lax.fori_loop(0, 16, rbody, jnp.zeros((16,), jnp.int32))
        gt = jnp.sum(gsum)
        nxt = cum + gt
        cross_a = (1 - fa) * jnp.where(nxt >= k_a, 1, 0)
        ga = jnp.where(cross_a == 1, g, ga)
        ca = jnp.where(cross_a == 1, cum, ca)
        cross_b = (1 - fb) * jnp.where(nxt >= k_b, 1, 0)
        gb = jnp.where(cross_b == 1, g, gb)
        cb = jnp.where(cross_b == 1, cum, cb)
        return nxt, ga, ca, fa | cross_a, gb, cb, fb | cross_b

    _, ga, ca, _, gb, cb, _ = lax.fori_loop(
        0, ngroups, gbody, (zero, zero, zero, zero, zero, zero, zero))

    def drill(gstar, cbase, k):
        def jbody(j, carry):
            cum, bsel, cbel, found = carry
            t = jnp.sum(ref[base + row_of(gstar * 16 + j)])
            crossed = (1 - found) * jnp.where(cum + t >= k, 1, 0)
            bsel = jnp.where(crossed == 1, j, bsel)
            cbel = jnp.where(crossed == 1, cum, cbel)
            return cum + t, bsel, cbel, found | crossed

        _, jstar, cbel, _ = lax.fori_loop(0, 16, jbody,
                                          (cbase, zero, cbase, zero))
        return row_of(gstar * 16 + jstar), cbel

    ba, ca2 = drill(ga, ca, k_a)
    bb, cb2 = drill(gb, cb, k_b)
    return ba, ca2, bb, cb2


def _sc_select_body(x_hbm, out_hbm, buf0, buf1, hist_a, hist_b, hist_c, outv,
                    sem0, sem1):
    w = lax.axis_index("s") * 2 + lax.axis_index("c")
    lanes = lax.iota(jnp.int32, 16)
    ones = jnp.full((16,), 1, jnp.int32)

    def zero_rows(ref, nrows):
        def zb(b, c):
            for u in range(_UNROLL):
                ref[b * _UNROLL + u] = jnp.zeros((16,), jnp.int32)
            return c
        lax.fori_loop(0, nrows // _UNROLL, zb, 0)

    def data_pass(process_vregs, unroll=_UNROLL):
        def inner(buf):
            def vbody(i, c2):
                base = i * 16 * unroll
                vs = [buf[pl.ds(base + u * 16, 16)] for u in range(unroll)]
                process_vregs(vs)
                return c2
            lax.fori_loop(0, _CH // (16 * unroll), vbody, 0)

        def wait(buf, sem):
            pltpu.make_async_copy(x_hbm.at[w, pl.ds(0, _CH)], buf, sem).wait()

        pltpu.async_copy(x_hbm.at[w, pl.ds(0, _CH)], buf0, sem0)

        def super_body(sg, carry):
            g0 = sg * 2
            pltpu.async_copy(x_hbm.at[w, pl.ds((g0 + 1) * _CH, _CH)],
                             buf1, sem1)
            wait(buf0, sem0)
            inner(buf0)

            @pl.when(sg < _NSUP - 1)
            def _():
                pltpu.async_copy(x_hbm.at[w, pl.ds((g0 + 2) * _CH, _CH)],
                                 buf0, sem0)

            wait(buf1, sem1)
            inner(buf1)
            return carry

        lax.fori_loop(0, _NSUP, super_body, 0)

    zero_rows(hist_a, _NB1)
    zero_rows(hist_b, _NB1)
    zero_rows(hist_c, _NB1)

    def p1(vs):
        b1s = [lax.shift_right_logical(v, 21) for v in vs]
        for b1 in b1s:
            plsc.addupdate_scatter(hist_a, [b1, lanes], ones)

    data_pass(p1, unroll=16)

    def l1_row(l):
        return jnp.where(l < 1024, 2047 - l, l - 1024)

    b1_lo, c1_lo, b1_hi, c1_hi = _scan_hist2(
        hist_a, 0, _NB1, jnp.int32(_KLO), jnp.int32(_KHI), l1_row)
    r_lo = jnp.int32(_KLO) - c1_lo
    r_hi = jnp.int32(_KHI) - c1_hi
    neg_lo = jnp.where(b1_lo >= 1024, 1, 0)
    neg_hi = jnp.where(b1_hi >= 1024, 1, 0)

    zero_rows(hist_a, _NB1)

    def p2(vs):
        b1s = [lax.shift_right_logical(v, 21) for v in vs]
        b2s = [lax.shift_right_logical(v, 10) & jnp.int32(0x7FF) for v in vs]
        m_los = [b1 == b1_lo for b1 in b1s]
        m_his = [b1 == b1_hi for b1 in b1s]
        for b2, m in zip(b2s, m_los):
            plsc.addupdate_scatter(hist_b, [b2, lanes], ones, mask=m)
        for b2, m in zip(b2s, m_his):
            plsc.addupdate_scatter(hist_c, [b2, lanes], ones, mask=m)

    data_pass(p2)

    def flip_row(neg, nrows):
        def row_of(l):
            return jnp.where(neg == 1, nrows - 1 - l, l)
        return row_of

    b2_lo, c2_lo, _, _ = _scan_hist2(hist_b, 0, _NB1, r_lo, r_lo,
                                     flip_row(neg_lo, _NB1))
    b2_hi, c2_hi, _, _ = _scan_hist2(hist_c, 0, _NB1, r_hi, r_hi,
                                     flip_row(neg_hi, _NB1))
    r2_lo = r_lo - c2_lo
    r2_hi = r_hi - c2_hi

    p22_lo = b1_lo * 2048 + b2_lo
    p22_hi = b1_hi * 2048 + b2_hi

    def p3(vs):
        p22s = [lax.shift_right_logical(v, 10) for v in vs]
        b3s = [v & jnp.int32(0x3FF) for v in vs]
        m_los = [p22 == p22_lo for p22 in p22s]
        m_his = [p22 == p22_hi for p22 in p22s]
        for b3, m in zip(b3s, m_los):
            plsc.addupdate_scatter(hist_a, [b3, lanes], ones, mask=m)
        for b3, m in zip(b3s, m_his):
            plsc.addupdate_scatter(hist_a, [b3 + jnp.int32(_NB3), lanes],
                                   ones, mask=m)

    data_pass(p3)

    b3_lo, _, _, _ = _scan_hist2(hist_a, 0, _NB3, r2_lo, r2_lo,
                                 flip_row(neg_lo, _NB3))
    b3_hi, _, _, _ = _scan_hist2(hist_a, _NB3, _NB3, r2_hi, r2_hi,
                                 flip_row(neg_hi, _NB3))

    raw_lo = (p22_lo << 10) | b3_lo
    raw_hi = (p22_hi << 10) | b3_hi
    key_lo = jnp.where(raw_lo >= 0, raw_lo, raw_lo ^ jnp.int32(0x7FFFFFFF))
    key_hi = jnp.where(raw_hi >= 0, raw_hi, raw_hi ^ jnp.int32(0x7FFFFFFF))

    outv[...] = jnp.where(lanes == 0, key_lo,
                          jnp.where(lanes == 1, key_hi, jnp.int32(0)))
    pltpu.sync_copy(outv, out_hbm.at[w])


def _label_body(thr_ref, x_ref, out_ref):
    i = pl.program_id(0)
    qlo = thr_ref[i, 0]
    qhi = thr_ref[i, 1]
    qlo = jnp.where(qlo == jnp.int32(-1), jnp.int32(0), qlo)
    qhi = jnp.where(qhi == jnp.int32(-1), jnp.int32(0), qhi)

    x = x_ref[0, 0]
    s = lax.bitcast_convert_type(x, jnp.int32)
    key = jnp.where(s >= 0, s, s ^ jnp.int32(0x7FFFFFFF))
    key = jnp.where(s == jnp.int32(_INT_MIN), jnp.int32(0), key)

    m_lo = jnp.sum((key < qlo).astype(jnp.int32))
    m_hi = jnp.sum((key < qhi).astype(jnp.int32))

    eq_lo = (key == qlo)
    eq_hi = (key == qhi)

    r_iota = lax.broadcasted_iota(jnp.int32, (_H, _W), 0)
    c_iota = lax.broadcasted_iota(jnp.int32, (_H, _W), 1)
    upper = (r_iota < c_iota).astype(jnp.bfloat16)
    lower = (c_iota < r_iota).astype(jnp.float32)

    dn = (((1,), (0,)), ((), ()))

    def prefix_of(eq):
        eq_b = eq.astype(jnp.bfloat16)
        within = lax.dot_general(eq_b, upper, dn,
                                 preferred_element_type=jnp.float32)
        rowsum = jnp.sum(eq.astype(jnp.float32), axis=1, keepdims=True)
        row_prefix = lax.dot_general(lower, rowsum, dn,
                                     preferred_element_type=jnp.float32)
        return within + row_prefix

    pref_lo = prefix_of(eq_lo)
    pref_hi = prefix_of(eq_hi)

    t_lo = (_KLO - m_lo).astype(jnp.float32)
    t_hi = (jnp.int32(_N) - m_hi - jnp.int32(_KMAX)).astype(jnp.float32)
    bg = (key < qlo) | (eq_lo & (pref_lo < t_lo))
    fg = (key > qhi) | (eq_hi & (pref_hi >= t_hi))

    out = jnp.where(fg, jnp.int32(1),
                    jnp.where(bg, jnp.int32(0), jnp.int32(_IGNORE)))
    out_ref[0] = out


def _sc_select(xf):
    mesh = plsc.VectorSubcoreMesh(core_axis_name="c", subcore_axis_name="s")
    return pl.kernel(
        _sc_select_body,
        out_type=jax.ShapeDtypeStruct((_NW, 16), jnp.int32),
        mesh=mesh,
        compiler_params=pltpu.CompilerParams(needs_layout_passes=False,
                                             use_tc_tiling_on_sc=False),
        scratch_types=[
            pltpu.VMEM((_CH,), jnp.int32),
            pltpu.VMEM((_CH,), jnp.int32),
            pltpu.VMEM((_NB1, 16), jnp.int32),
            pltpu.VMEM((_NB1, 16), jnp.int32),
            pltpu.VMEM((_NB1, 16), jnp.int32),
            pltpu.VMEM((16,), jnp.int32),
            pltpu.SemaphoreType.DMA,
            pltpu.SemaphoreType.DMA,
        ],
    )(xf)


def _tc_label(thr, x):
    b = x.shape[0]
    grid_spec = pl.GridSpec(
        grid=(b,),
        in_specs=[
            pl.BlockSpec(memory_space=pltpu.SMEM),
            pl.BlockSpec((1, 1, _H, _W), lambda i: (i, 0, 0, 0)),
        ],
        out_specs=pl.BlockSpec((1, _H, _W), lambda i: (i, 0, 0)),
    )
    return pl.pallas_call(
        _label_body,
        grid_spec=grid_spec,
        out_shape=jax.ShapeDtypeStruct((b, _H, _W), jnp.int32),
    )(thr, x)


@jax.jit
def kernel(x):
    b = x.shape[0]
    xf = lax.bitcast_convert_type(x.reshape(b, _N), jnp.int32)
    thr = _sc_select(xf)
    return _tc_label(thr, x)
